# baseline (speedup 1.0000x reference)
import functools

import jax
import jax.numpy as jnp
from jax import lax
from jax.experimental import pallas as pl
from jax.experimental.pallas import tpu as pltpu
from jax.experimental.pallas import tpu_sc as plsc

PAD_ID = 1
EPS = 1e-5
NC, NS = 2, 16
NW = NC * NS
SC_CHUNK = 64


def _sc_word_gather(idx_flat, table):
    n = idx_flat.shape[0]
    d = table.shape[1]
    b_per_w = n // NW
    nch = b_per_w // SC_CHUNK
    mesh = plsc.VectorSubcoreMesh(core_axis_name="c", subcore_axis_name="s")

    @functools.partial(
        pl.kernel,
        out_type=jax.ShapeDtypeStruct((n, d), jnp.float32),
        mesh=mesh,
        scratch_types=[
            pltpu.VMEM((b_per_w,), jnp.int32),
            pltpu.VMEM((SC_CHUNK, d), jnp.float32),
            pltpu.VMEM((SC_CHUNK, d), jnp.float32),
            pltpu.SemaphoreType.DMA,
            pltpu.SemaphoreType.DMA,
            pltpu.SemaphoreType.DMA,
            pltpu.SemaphoreType.DMA,
        ],
    )
    def gather_kernel(idx_hbm, table_hbm, out_hbm,
                      idx_v, rows0, rows1, gs0, gs1, ws0, ws1):
        wid = lax.axis_index("s") * NC + lax.axis_index("c")
        base = wid * b_per_w
        rows, gs, ws = [rows0, rows1], [gs0, gs1], [ws0, ws1]
        pltpu.sync_copy(idx_hbm.at[pl.ds(base, b_per_w)], idx_v)

        gath, wb = {}, {}
        gath[0] = pltpu.async_copy(
            table_hbm.at[idx_v.at[pl.ds(0, SC_CHUNK)]], rows[0], gs[0])
        for i in range(nch):
            b = i % 2
            if i + 1 < nch:
                nb = (i + 1) % 2
                if i >= 1:
                    wb[i - 1].wait()
                gath[i + 1] = pltpu.async_copy(
                    table_hbm.at[idx_v.at[pl.ds((i + 1) * SC_CHUNK, SC_CHUNK)]],
                    rows[nb], gs[nb])
            gath[i].wait()
            wb[i] = pltpu.async_copy(
                rows[b], out_hbm.at[pl.ds(base + i * SC_CHUNK, SC_CHUNK)],
                ws[b])
        if nch >= 2:
            wb[nch - 2].wait()
        wb[nch - 1].wait()

    return gather_kernel(idx_flat, table)


def _tc_finish(x_col, h_word, tri, w_pos_b, gamma, beta, out_buf, row0, B):
    Bg, S, _ = x_col.shape
    P, D = w_pos_b.shape
    inv_d = 1.0 / D

    def body(x_ref, hw_ref, tri_ref, wpos_ref, g_ref, b_ref, *rest):
        out_ref = rest[-1]
        xc = x_ref[0]
        maskf = (xc != PAD_ID).astype(jnp.float32)
        cnt = jnp.dot(tri_ref[...], maskf, preferred_element_type=jnp.float32)
        pos = cnt * maskf + 1.0
        pos_i = pos.astype(jnp.int32)
        ph = lax.broadcasted_iota(jnp.int32, (S, P), 1)
        oh = (pos_i == ph).astype(jnp.bfloat16)
        pe = jnp.dot(oh, wpos_ref[...], preferred_element_type=jnp.float32)
        h = hw_ref[0] + pe
        mean = jnp.sum(h, axis=-1, keepdims=True) * inv_d
        msq = jnp.sum(h * h, axis=-1, keepdims=True) * inv_d
        s = lax.rsqrt(msq - mean * mean + EPS)
        out_ref[0] = ((h - mean) * s) * g_ref[...] + b_ref[...]

    in_specs = [
        pl.BlockSpec((1, S, 1), lambda b: (b, 0, 0)),
        pl.BlockSpec((1, S, D), lambda b: (b, 0, 0)),
        pl.BlockSpec((S, S), lambda b: (0, 0)),
        pl.BlockSpec((P, D), lambda b: (0, 0)),
        pl.BlockSpec((1, D), lambda b: (0, 0)),
        pl.BlockSpec((1, D), lambda b: (0, 0)),
    ]
    args = [x_col, h_word, tri, w_pos_b, gamma, beta]
    aliases = {}
    if out_buf is not None:
        in_specs.append(pl.BlockSpec(memory_space=pl.ANY))
        args.append(out_buf)
        aliases = {len(args) - 1: 0}
    blk0 = row0 // 1
    return pl.pallas_call(
        body,
        grid=(Bg,),
        in_specs=in_specs,
        out_specs=pl.BlockSpec((1, S, D), lambda b: (blk0 + b, 0, 0)),
        out_shape=jax.ShapeDtypeStruct((B, S, D), jnp.float32),
        input_output_aliases=aliases,
    )(*args)


N_GROUPS = 2


def kernel(x, W_word, W_pos, W_type, gamma, beta):
    B, S = x.shape
    D = W_word.shape[1]
    xi = x.astype(jnp.int32)
    r = lax.broadcasted_iota(jnp.int32, (S, S), 0)
    c = lax.broadcasted_iota(jnp.int32, (S, S), 1)
    tri = (c <= r).astype(jnp.float32)
    P = W_pos.shape[0]
    P_pad = ((P + 15) // 16) * 16
    w_pos_b = jnp.pad((W_pos + W_type[0]).astype(jnp.bfloat16),
                      ((0, P_pad - P), (0, 0)))
    g2 = gamma.reshape(1, D)
    b2 = beta.reshape(1, D)
    x_col = xi.reshape(B, S, 1)

    Bg = B // N_GROUPS
    out = None
    for g in range(N_GROUPS):
        sl = slice(g * Bg, (g + 1) * Bg)
        hw_g = _sc_word_gather(
            xi[sl].reshape(Bg * S), W_word).reshape(Bg, S, D)
        out = _tc_finish(x_col[sl], hw_g, tri, w_pos_b, g2, b2,
                         out, g * Bg, B)
    return out


# speedup vs baseline: 1.0409x; 1.0409x over previous
import functools

import jax
import jax.numpy as jnp
from jax import lax
from jax.experimental import pallas as pl
from jax.experimental.pallas import tpu as pltpu
from jax.experimental.pallas import tpu_sc as plsc

PAD_ID = 1
EPS = 1e-5
NC, NS = 2, 16
NW = NC * NS
SC_CHUNK = 128


def _sc_word_gather(idx_flat, table):
    n = idx_flat.shape[0]
    d = table.shape[1]
    b_per_w = n // NW
    nch = b_per_w // SC_CHUNK
    mesh = plsc.VectorSubcoreMesh(core_axis_name="c", subcore_axis_name="s")

    @functools.partial(
        pl.kernel,
        out_type=jax.ShapeDtypeStruct((n, d), jnp.float32),
        mesh=mesh,
        scratch_types=[
            pltpu.VMEM((b_per_w,), jnp.int32),
            pltpu.VMEM((SC_CHUNK, d), jnp.float32),
            pltpu.SemaphoreType.DMA,
        ],
    )
    def gather_kernel(idx_hbm, table_hbm, out_hbm, idx_v, rows_v, sem):
        wid = lax.axis_index("s") * NC + lax.axis_index("c")
        base = wid * b_per_w
        pltpu.sync_copy(idx_hbm.at[pl.ds(base, b_per_w)], idx_v)

        def body(i, carry):
            off = i * SC_CHUNK
            pltpu.async_copy(
                table_hbm.at[idx_v.at[pl.ds(off, SC_CHUNK)]], rows_v,
                sem).wait()
            pltpu.sync_copy(rows_v, out_hbm.at[pl.ds(base + off, SC_CHUNK)])
            return carry

        lax.fori_loop(0, nch, body, 0)

    return gather_kernel(idx_flat, table)


def _tc_finish(x_col, h_word, tri, w_pos_b, gamma, beta, out_buf, row0, B):
    Bg, S, _ = x_col.shape
    P, D = w_pos_b.shape
    inv_d = 1.0 / D

    def body(x_ref, hw_ref, tri_ref, wpos_ref, g_ref, b_ref, *rest):
        out_ref = rest[-1]
        xc = x_ref[0]
        maskf = (xc != PAD_ID).astype(jnp.float32)
        cnt = jnp.dot(tri_ref[...], maskf, preferred_element_type=jnp.float32)
        pos = cnt * maskf + 1.0
        pos_i = pos.astype(jnp.int32)
        ph = lax.broadcasted_iota(jnp.int32, (S, P), 1)
        oh = (pos_i == ph).astype(jnp.bfloat16)
        pe = jnp.dot(oh, wpos_ref[...], preferred_element_type=jnp.float32)
        h = hw_ref[0] + pe
        mean = jnp.sum(h, axis=-1, keepdims=True) * inv_d
        msq = jnp.sum(h * h, axis=-1, keepdims=True) * inv_d
        s = lax.rsqrt(msq - mean * mean + EPS)
        out_ref[0] = ((h - mean) * s) * g_ref[...] + b_ref[...]

    in_specs = [
        pl.BlockSpec((1, S, 1), lambda b: (b, 0, 0)),
        pl.BlockSpec((1, S, D), lambda b: (b, 0, 0)),
        pl.BlockSpec((S, S), lambda b: (0, 0)),
        pl.BlockSpec((P, D), lambda b: (0, 0)),
        pl.BlockSpec((1, D), lambda b: (0, 0)),
        pl.BlockSpec((1, D), lambda b: (0, 0)),
    ]
    args = [x_col, h_word, tri, w_pos_b, gamma, beta]
    aliases = {}
    if out_buf is not None:
        in_specs.append(pl.BlockSpec(memory_space=pl.ANY))
        args.append(out_buf)
        aliases = {len(args) - 1: 0}
    blk0 = row0 // 1
    return pl.pallas_call(
        body,
        grid=(Bg,),
        in_specs=in_specs,
        out_specs=pl.BlockSpec((1, S, D), lambda b: (blk0 + b, 0, 0)),
        out_shape=jax.ShapeDtypeStruct((B, S, D), jnp.float32),
        input_output_aliases=aliases,
    )(*args)


N_GROUPS = 4


def kernel(x, W_word, W_pos, W_type, gamma, beta):
    B, S = x.shape
    D = W_word.shape[1]
    xi = x.astype(jnp.int32)
    r = lax.broadcasted_iota(jnp.int32, (S, S), 0)
    c = lax.broadcasted_iota(jnp.int32, (S, S), 1)
    tri = (c <= r).astype(jnp.float32)
    P = W_pos.shape[0]
    P_pad = ((P + 15) // 16) * 16
    w_pos_b = jnp.pad((W_pos + W_type[0]).astype(jnp.bfloat16),
                      ((0, P_pad - P), (0, 0)))
    g2 = gamma.reshape(1, D)
    b2 = beta.reshape(1, D)
    x_col = xi.reshape(B, S, 1)

    Bg = B // N_GROUPS
    out = None
    for g in range(N_GROUPS):
        sl = slice(g * Bg, (g + 1) * Bg)
        hw_g = _sc_word_gather(
            xi[sl].reshape(Bg * S), W_word).reshape(Bg, S, D)
        out = _tc_finish(x_col[sl], hw_g, tri, w_pos_b, g2, b2,
                         out, g * Bg, B)
    return out


# speedup vs baseline: 1.0591x; 1.0175x over previous
import functools

import jax
import jax.numpy as jnp
from jax import lax
from jax.experimental import pallas as pl
from jax.experimental.pallas import tpu as pltpu
from jax.experimental.pallas import tpu_sc as plsc

PAD_ID = 1
EPS = 1e-5
NC, NS = 2, 16
NW = NC * NS
SC_CHUNK = 128


def _sc_word_gather(idx_flat, table):
    n = idx_flat.shape[0]
    d = table.shape[1]
    b_per_w = n // NW
    nch = b_per_w // SC_CHUNK
    mesh = plsc.VectorSubcoreMesh(core_axis_name="c", subcore_axis_name="s")

    @functools.partial(
        pl.kernel,
        out_type=jax.ShapeDtypeStruct((n, d), jnp.float32),
        mesh=mesh,
        scratch_types=[
            pltpu.VMEM((b_per_w,), jnp.int32),
            pltpu.VMEM((SC_CHUNK, d), jnp.float32),
            pltpu.SemaphoreType.DMA,
        ],
    )
    def gather_kernel(idx_hbm, table_hbm, out_hbm, idx_v, rows_v, sem):
        wid = lax.axis_index("s") * NC + lax.axis_index("c")
        base = wid * b_per_w
        pltpu.sync_copy(idx_hbm.at[pl.ds(base, b_per_w)], idx_v)

        def body(i, carry):
            off = i * SC_CHUNK
            pltpu.async_copy(
                table_hbm.at[idx_v.at[pl.ds(off, SC_CHUNK)]], rows_v,
                sem).wait()
            pltpu.sync_copy(rows_v, out_hbm.at[pl.ds(base + off, SC_CHUNK)])
            return carry

        lax.fori_loop(0, nch, body, 0)

    return gather_kernel(idx_flat, table)


def _tc_finish(x_col, h_word, tri, w_pos_b, gamma, beta, out_buf, row0, B):
    Bg, S, _ = x_col.shape
    P, D = w_pos_b.shape
    inv_d = 1.0 / D

    def body(x_ref, hw_ref, tri_ref, wpos_ref, g_ref, b_ref, *rest):
        out_ref = rest[-1]
        xc = x_ref[0]
        maskf = (xc != PAD_ID).astype(jnp.float32)
        cnt = jnp.dot(tri_ref[...], maskf, preferred_element_type=jnp.float32)
        pos = cnt * maskf + 1.0
        pos_i = pos.astype(jnp.int32)
        ph = lax.broadcasted_iota(jnp.int32, (S, P), 1)
        oh = (pos_i == ph).astype(jnp.bfloat16)
        pe = jnp.dot(oh, wpos_ref[...], preferred_element_type=jnp.float32)
        h = hw_ref[0] + pe
        mean = jnp.sum(h, axis=-1, keepdims=True) * inv_d
        msq = jnp.sum(h * h, axis=-1, keepdims=True) * inv_d
        s = lax.rsqrt(msq - mean * mean + EPS)
        out_ref[0] = ((h - mean) * s) * g_ref[...] + b_ref[...]

    in_specs = [
        pl.BlockSpec((1, S, 1), lambda b: (b, 0, 0)),
        pl.BlockSpec((1, S, D), lambda b: (b, 0, 0)),
        pl.BlockSpec((S, S), lambda b: (0, 0)),
        pl.BlockSpec((P, D), lambda b: (0, 0)),
        pl.BlockSpec((1, D), lambda b: (0, 0)),
        pl.BlockSpec((1, D), lambda b: (0, 0)),
    ]
    args = [x_col, h_word, tri, w_pos_b, gamma, beta]
    aliases = {}
    if out_buf is not None:
        in_specs.append(pl.BlockSpec(memory_space=pl.ANY))
        args.append(out_buf)
        aliases = {len(args) - 1: 0}
    blk0 = row0 // 1
    return pl.pallas_call(
        body,
        grid=(Bg,),
        in_specs=in_specs,
        out_specs=pl.BlockSpec((1, S, D), lambda b: (blk0 + b, 0, 0)),
        out_shape=jax.ShapeDtypeStruct((B, S, D), jnp.float32),
        input_output_aliases=aliases,
    )(*args)


N_GROUPS = 4


def kernel(x, W_word, W_pos, W_type, gamma, beta):
    B, S = x.shape
    D = W_word.shape[1]
    xi = x.astype(jnp.int32)
    r = lax.broadcasted_iota(jnp.int32, (S, S), 0)
    c = lax.broadcasted_iota(jnp.int32, (S, S), 1)
    tri = (c <= r).astype(jnp.float32)
    P = W_pos.shape[0]
    P_pad = ((P + 15) // 16) * 16
    w_pos_b = jnp.pad((W_pos + W_type[0]).astype(jnp.bfloat16),
                      ((0, P_pad - P), (0, 0)))
    g2 = gamma.reshape(1, D)
    b2 = beta.reshape(1, D)
    x_col = xi.reshape(B, S, 1)

    sizes = [16, 32, 40, 40]
    out = None
    row = 0
    for bg in sizes:
        sl = slice(row, row + bg)
        hw_g = _sc_word_gather(
            xi[sl].reshape(bg * S), W_word).reshape(bg, S, D)
        out = _tc_finish(x_col[sl], hw_g, tri, w_pos_b, g2, b2,
                         out, row, B)
        row += bg
    return out
